# baseline (speedup 1.0000x reference)
import functools

import jax
import jax.numpy as jnp
from jax import lax
from jax.experimental import pallas as pl
from jax.experimental.pallas import tpu as pltpu
from jax.experimental.pallas import tpu_sc as plsc

_VOCAB = 1000
_B = 1024
_T = 50
_TCHUNK = _T // 2
_NV = 64
_NQ = 4
_QV = _NV // _NQ
_LANES = 16
_KB = _B // _LANES
_VT = _VOCAB // 8
_BT = _B // 128


def _make_gather():
    mesh = plsc.VectorSubcoreMesh(core_axis_name="c", subcore_axis_name="s")

    @functools.partial(
        pl.kernel,
        mesh=mesh,
        compiler_params=pltpu.CompilerParams(
            use_tc_tiling_on_sc=False, needs_layout_passes=False),
        out_type=jax.ShapeDtypeStruct((_T, _VT, _BT, 8, 128), jnp.float32),
        scratch_types=[
            pltpu.VMEM((_NV * _VOCAB,), jnp.float32),
            pltpu.VMEM((_B,), jnp.int32),
            pltpu.VMEM((_QV // 8, _BT, 8, 128), jnp.float32),
            pltpu.VMEM((_QV // 8, _BT, 8, 128), jnp.float32),
            pltpu.SemaphoreType.DMA,
            pltpu.SemaphoreType.DMA,
            pltpu.SemaphoreType.DMA,
        ],
    )
    def gather_kernel(idxt_hbm, wt_hbm, out_hbm, wt_v, idx_v, bufa, bufb,
                      sem, sa, sb):
        cid = lax.axis_index("c")
        sid = lax.axis_index("s")
        t0 = cid * _TCHUNK
        v0 = jnp.minimum(sid * _NV, _VOCAB - _NV)
        vt0 = v0 // 8

        pltpu.async_copy(wt_hbm.at[pl.ds(v0 * _VOCAB, _NV * _VOCAB)],
                         wt_v, sem).wait()

        bufs = (bufa, bufb)
        sems = (sa, sb)

        def compute_quarter(q, buf):
            @plsc.parallel_loop(0, _KB, 1, unroll=16)
            def _(k):
                bt = k // 8
                co = (k % 8) * _LANES
                iv = idx_v[pl.ds(k * _LANES, _LANES)]
                for vl in range(_QV):
                    voff = (q * _QV + vl) * _VOCAB
                    vals = plsc.load_gather(wt_v, [iv + voff])
                    buf[vl // 8, bt, vl % 8, pl.ds(co, _LANES)] = vals

        def dst(t, q):
            return out_hbm.at[t0 + t, pl.ds(vt0 + q * (_QV // 8), _QV // 8)]

        def t_body(tl, _):
            pltpu.async_copy(idxt_hbm.at[t0 + tl], idx_v, sem).wait()
            for q in range(_NQ):
                buf = bufs[q % 2]
                s = sems[q % 2]
                if q >= 2:
                    pltpu.make_async_copy(buf, dst(tl, q - 2), s).wait()
                else:
                    @pl.when(tl > 0)
                    def _():
                        pltpu.make_async_copy(buf, dst(tl - 1, q), s).wait()
                compute_quarter(q, buf)
                pltpu.async_copy(buf, dst(tl, q), s)
            return 0

        lax.fori_loop(0, _TCHUNK, t_body, 0)
        pltpu.make_async_copy(bufa, dst(_TCHUNK - 1, 2), sa).wait()
        pltpu.make_async_copy(bufb, dst(_TCHUNK - 1, 3), sb).wait()

    return gather_kernel


_gather = _make_gather()


def kernel(idx, W):
    idxt = idx.T.astype(jnp.int32)
    wt = W.T.reshape(-1)
    p5 = _gather(idxt, wt)
    p = p5.transpose(0, 1, 3, 2, 4).reshape(_T, _VOCAB, _B)
    return jnp.transpose(p, (2, 0, 1))


# speedup vs baseline: 1.8502x; 1.8502x over previous
import functools

import jax
import jax.numpy as jnp
from jax import lax
from jax.experimental import pallas as pl
from jax.experimental.pallas import tpu as pltpu
from jax.experimental.pallas import tpu_sc as plsc

_VOCAB = 1000
_B = 1024
_T = 50
_TCHUNK = _T // 2
_NV = 64
_NQ = 4
_QV = _NV // _NQ
_LANES = 16
_KB = _B // _LANES
_VT = _VOCAB // 8
_BT = _B // 128


def _make_gather():
    mesh = plsc.VectorSubcoreMesh(core_axis_name="c", subcore_axis_name="s")

    @functools.partial(
        pl.kernel,
        mesh=mesh,
        compiler_params=pltpu.CompilerParams(
            use_tc_tiling_on_sc=False, needs_layout_passes=False),
        out_type=jax.ShapeDtypeStruct((_T, _VT, _BT, 8, 128), jnp.float32),
        scratch_types=[
            pltpu.VMEM((_NV * _VOCAB,), jnp.float32),
            pltpu.VMEM((_B,), jnp.int32),
            pltpu.VMEM((_QV // 8, _BT, 8, 128), jnp.float32),
            pltpu.VMEM((_QV // 8, _BT, 8, 128), jnp.float32),
            pltpu.SemaphoreType.DMA,
            pltpu.SemaphoreType.DMA,
            pltpu.SemaphoreType.DMA,
        ],
    )
    def gather_kernel(idxt_hbm, wt_hbm, out_hbm, wt_v, idx_v, bufa, bufb,
                      sem, sa, sb):
        cid = lax.axis_index("c")
        sid = lax.axis_index("s")
        t0 = cid * _TCHUNK
        v0 = jnp.minimum(sid * _NV, _VOCAB - _NV)
        vt0 = v0 // 8

        pltpu.async_copy(wt_hbm.at[pl.ds(v0 * _VOCAB, _NV * _VOCAB)],
                         wt_v, sem).wait()

        bufs = (bufa, bufb)
        sems = (sa, sb)

        def compute_quarter(q, buf):
            @plsc.parallel_loop(0, _KB, 1, unroll=4)
            def _(k):
                bt = k // 8
                co = (k % 8) * _LANES
                iv = idx_v[pl.ds(k * _LANES, _LANES)]
                for vl in range(_QV):
                    voff = (q * _QV + vl) * _VOCAB
                    vals = plsc.load_gather(wt_v, [iv + voff])
                    buf[vl // 8, bt, vl % 8, pl.ds(co, _LANES)] = vals

        def dst(t, q):
            return out_hbm.at[t0 + t, pl.ds(vt0 + q * (_QV // 8), _QV // 8)]

        def t_body(tl, _):
            pltpu.async_copy(idxt_hbm.at[t0 + tl], idx_v, sem).wait()
            for q in range(_NQ):
                buf = bufs[q % 2]
                s = sems[q % 2]
                if q >= 2:
                    pltpu.make_async_copy(buf, dst(tl, q - 2), s).wait()
                else:
                    @pl.when(tl > 0)
                    def _():
                        pltpu.make_async_copy(buf, dst(tl - 1, q), s).wait()
                compute_quarter(q, buf)
                pltpu.async_copy(buf, dst(tl, q), s)
            return 0

        lax.fori_loop(0, _TCHUNK, t_body, 0)
        pltpu.make_async_copy(bufa, dst(_TCHUNK - 1, 2), sa).wait()
        pltpu.make_async_copy(bufb, dst(_TCHUNK - 1, 3), sb).wait()

    return gather_kernel


_gather = _make_gather()


def kernel(idx, W):
    idxt = idx.T.astype(jnp.int32)
    wt = W.T.reshape(-1)
    p5 = _gather(idxt, wt)
    p = p5.transpose(0, 1, 3, 2, 4).reshape(_T, _VOCAB, _B)
    return jnp.transpose(p, (2, 0, 1))


# speedup vs baseline: 1.8887x; 1.0208x over previous
import functools

import jax
import jax.numpy as jnp
from jax import lax
from jax.experimental import pallas as pl
from jax.experimental.pallas import tpu as pltpu
from jax.experimental.pallas import tpu_sc as plsc

_VOCAB = 1000
_B = 1024
_T = 50
_TCHUNK = _T // 2
_NV = 64
_NQ = 4
_QV = _NV // _NQ
_LANES = 16
_KB = _B // _LANES
_VT = _VOCAB // 8
_BT = _B // 128


def _make_gather():
    mesh = plsc.VectorSubcoreMesh(core_axis_name="c", subcore_axis_name="s")

    @functools.partial(
        pl.kernel,
        mesh=mesh,
        compiler_params=pltpu.CompilerParams(
            use_tc_tiling_on_sc=False, needs_layout_passes=False),
        out_type=jax.ShapeDtypeStruct((_T, _VT, _BT, 8, 128), jnp.float32),
        scratch_types=[
            pltpu.VMEM((_NV * _VOCAB,), jnp.float32),
            pltpu.VMEM((_B,), jnp.int32),
            pltpu.VMEM((_QV // 8, _BT, 8, 128), jnp.float32),
            pltpu.VMEM((_QV // 8, _BT, 8, 128), jnp.float32),
            pltpu.SemaphoreType.DMA,
            pltpu.SemaphoreType.DMA,
            pltpu.SemaphoreType.DMA,
        ],
    )
    def gather_kernel(idxt_hbm, wt_hbm, out_hbm, wt_v, idx_v, bufa, bufb,
                      sem, sa, sb):
        cid = lax.axis_index("c")
        sid = lax.axis_index("s")
        t0 = cid * _TCHUNK
        v0 = jnp.minimum(sid * _NV, _VOCAB - _NV)
        vt0 = v0 // 8

        pltpu.async_copy(wt_hbm.at[pl.ds(v0 * _VOCAB, _NV * _VOCAB)],
                         wt_v, sem).wait()

        bufs = (bufa, bufb)
        sems = (sa, sb)

        def compute_quarter(q, buf):
            @plsc.parallel_loop(0, _KB, 1, unroll=8)
            def _(k):
                bt = k // 8
                co = (k % 8) * _LANES
                iv = idx_v[pl.ds(k * _LANES, _LANES)]
                for vl in range(_QV):
                    voff = (q * _QV + vl) * _VOCAB
                    vals = plsc.load_gather(wt_v, [iv + voff])
                    buf[vl // 8, bt, vl % 8, pl.ds(co, _LANES)] = vals

        def dst(t, q):
            return out_hbm.at[t0 + t, pl.ds(vt0 + q * (_QV // 8), _QV // 8)]

        def t_body(tl, _):
            pltpu.async_copy(idxt_hbm.at[t0 + tl], idx_v, sem).wait()
            for q in range(_NQ):
                buf = bufs[q % 2]
                s = sems[q % 2]
                if q >= 2:
                    pltpu.make_async_copy(buf, dst(tl, q - 2), s).wait()
                else:
                    @pl.when(tl > 0)
                    def _():
                        pltpu.make_async_copy(buf, dst(tl - 1, q), s).wait()
                compute_quarter(q, buf)
                pltpu.async_copy(buf, dst(tl, q), s)
            return 0

        lax.fori_loop(0, _TCHUNK, t_body, 0)
        pltpu.make_async_copy(bufa, dst(_TCHUNK - 1, 2), sa).wait()
        pltpu.make_async_copy(bufb, dst(_TCHUNK - 1, 3), sb).wait()

    return gather_kernel


_gather = _make_gather()


def kernel(idx, W):
    idxt = idx.T.astype(jnp.int32)
    wt = W.T.reshape(-1)
    p5 = _gather(idxt, wt)
    p = p5.transpose(0, 1, 3, 2, 4).reshape(_T, _VOCAB, _B)
    return jnp.transpose(p, (2, 0, 1))
